# Initial kernel scaffold; baseline (speedup 1.0000x reference)
import jax
import jax.numpy as jnp
from jax.experimental import pallas as pl


def kernel(invariant_node, invariant_edge, distances, distance_vector, neighbors, neighbor_mask, equivariant_node_F, equivariant_node_f, equivariant_node_dr, W_ime, b_ime, W_imn1, b_imn1, W_imn2, b_imn2, W_emc, W_emf1, b_emf1, W_emf2, b_emf2, W_esc1, b_esc1, W_esc2, b_esc2, W_eme1, W_eme2, W_isc1, b_isc1, W_isc2, b_isc2):
    raise NotImplementedError("write your pallas kernel here")


# speedup vs baseline: 4.7807x; 4.7807x over previous
import jax
import jax.numpy as jnp
from jax.experimental import pallas as pl
from jax.experimental.pallas import tpu as pltpu
from jax.experimental.pallas import tpu_sc as plsc

NF = 128
NN = 16
CUTOFF = 5.0
F32 = jnp.float32


def _dot(a, b):
    return jnp.dot(a, b, preferred_element_type=F32)


def _imn_body(inv_ref, w1_ref, b1_ref, w2_ref, b2_ref, out_ref):
    h = jax.nn.silu(_dot(inv_ref[...], w1_ref[...]) + b1_ref[...])
    out_ref[...] = _dot(h, w2_ref[...]) + b2_ref[...]


def _imn_call(inv, w1, b1, w2, b2, block, interpret=False):
    a = inv.shape[0]
    grid = (a // block,)
    return pl.pallas_call(
        _imn_body,
        grid=grid,
        in_specs=[
            pl.BlockSpec((block, NF), lambda i: (i, 0)),
            pl.BlockSpec((NF, NF), lambda i: (0, 0)),
            pl.BlockSpec((1, NF), lambda i: (0, 0)),
            pl.BlockSpec((NF, NF), lambda i: (0, 0)),
            pl.BlockSpec((1, NF), lambda i: (0, 0)),
        ],
        out_specs=pl.BlockSpec((block, NF), lambda i: (i, 0)),
        out_shape=jax.ShapeDtypeStruct((a, NF), F32),
        interpret=interpret,
    )(inv, w1, b1, w2, b2)


def _gather_rows(table, flat_idx, value_dim, window):
    num_idx = flat_idx.shape[0]
    idx2 = flat_idx.reshape(1, num_idx)
    mesh = plsc.VectorSubcoreMesh(core_axis_name="c", subcore_axis_name="s")

    @pl.kernel(
        out_type=jax.ShapeDtypeStruct((num_idx, value_dim), table.dtype),
        mesh=mesh,
    )
    def k(x_hbm, i_hbm, o_hbm):
        def body(i_vmem, o_vmem):
            pltpu.sync_copy(x_hbm.at[i_vmem.at[0]], o_vmem)

        pltpu.emit_pipeline(
            body,
            grid=(num_idx // window,),
            in_specs=[pl.BlockSpec((1, window), lambda i: (0, i))],
            out_specs=[pl.BlockSpec((window, value_dim), lambda i: (i, 0))],
            core_axis_name=("c", "s"),
            dimension_semantics=(pltpu.PARALLEL,),
        )(i_hbm, o_hbm)

    return k(table, idx2)


def _main_body(inv_ref, edge_ref, dist_ref, dv_ref, mask_ref, drin_ref, fin_ref,
               imn_ref, gimn_ref, gdr_ref,
               wime_ref, bime_ref, wemc_ref,
               wemf1_ref, bemf1_ref, wemf2_ref, bemf2_ref,
               weme1_ref, weme2_ref,
               wesc1_ref, besc1_ref, wesc2_ref, besc2_ref,
               wisc1_ref, bisc1_ref, wisc2_ref, bisc2_ref,
               invout_ref, Fout_ref, fout_ref, drout_ref):
    silu = jax.nn.silu
    t = inv_ref.shape[0]

    ime = _dot(edge_ref[...], wime_ref[...]) + bime_ref[...]
    x = dist_ref[...] * (1.0 / CUTOFF)
    cut = jnp.where(x < 1.0, 1.0 - 6.0 * x**5 + 15.0 * x**4 - 10.0 * x**3, 0.0)

    imn = imn_ref[...]
    gimn = gimn_ref[...].reshape(t, NN, NF)
    msg = (ime.reshape(t, NN, NF) * cut[:, :, None]) * gimn * imn[:, None, :]

    mask = mask_ref[...]
    m3 = mask[:, :, None]
    inv_new = inv_ref[...] + jnp.sum(msg * m3, axis=1)

    msg2 = msg.reshape(t * NN, NF)
    h1 = _dot(silu(_dot(msg2, wemf1_ref[...]) + bemf1_ref[...]),
              wemf2_ref[...]) + bemf2_ref[...]
    h2 = _dot(silu(_dot(msg2, weme1_ref[...])), weme2_ref[...])
    emf_e = h1.reshape(t, NN, NF)
    eme_e = h2.reshape(t, NN, NF)

    emc = jnp.sum(msg * wemc_ref[...][None], axis=2)

    esc = _dot(silu(_dot(inv_new, wesc1_ref[...]) + besc1_ref[...]),
               wesc2_ref[...]) + besc2_ref[...]
    isc = _dot(silu(_dot(inv_new, wisc1_ref[...]) + bisc1_ref[...]),
               wisc2_ref[...]) + bisc2_ref[...]

    lane = jax.lax.broadcasted_iota(jnp.int32, (1, NF), 1)
    F_acc = jnp.zeros((t, NF), F32)
    dot_acc = jnp.zeros((t, NF), F32)
    for c in range(3):
        dv_c = dv_ref[c]
        emFm = emc * dv_c * mask
        F_acc += jnp.sum(emFm, axis=1, keepdims=True) * (lane == c).astype(F32)
        updf_c = jnp.sum(emf_e * emFm[:, :, None], axis=1)
        gdr_c = gdr_ref[:, :, c * NF:(c + 1) * NF]
        upddr_c = jnp.sum(eme_e * gdr_c * m3, axis=1)
        f_new_c = fin_ref[:, c, :] + updf_c
        dr_new_c = drin_ref[:, c, :] + upddr_c + esc * updf_c
        fout_ref[:, c, :] = f_new_c
        drout_ref[:, c, :] = dr_new_c
        dot_acc += f_new_c * dr_new_c

    invout_ref[...] = inv_new - isc * dot_acc
    Fout_ref[...] = F_acc


def _main_call(inv, edge, dist, dv3, mask, dr_in, f_in, imn, gimn, gdr,
               w_ime, b_ime, wemc_row,
               w_emf1, b_emf1, w_emf2, b_emf2, w_eme1, w_eme2,
               w_esc1, b_esc1, w_esc2, b_esc2, w_isc1, b_isc1, w_isc2, b_isc2,
               block, interpret=False):
    a = inv.shape[0]
    nb = edge.shape[1]
    grid = (a // block,)
    e_blk = block * NN

    def w_spec(shape):
        return pl.BlockSpec(shape, lambda i: tuple(0 for _ in shape))

    in_specs = [
        pl.BlockSpec((block, NF), lambda i: (i, 0)),
        pl.BlockSpec((e_blk, nb), lambda i: (i, 0)),
        pl.BlockSpec((block, NN), lambda i: (i, 0)),
        pl.BlockSpec((3, block, NN), lambda i: (0, i, 0)),
        pl.BlockSpec((block, NN), lambda i: (i, 0)),
        pl.BlockSpec((block, 3, NF), lambda i: (i, 0, 0)),
        pl.BlockSpec((block, 3, NF), lambda i: (i, 0, 0)),
        pl.BlockSpec((block, NF), lambda i: (i, 0)),
        pl.BlockSpec((e_blk, NF), lambda i: (i, 0)),
        pl.BlockSpec((block, NN, 3 * NF), lambda i: (i, 0, 0)),
        w_spec((nb, NF)), w_spec((1, NF)), w_spec((1, NF)),
        w_spec((NF, NF)), w_spec((1, NF)), w_spec((NF, NF)), w_spec((1, NF)),
        w_spec((NF, NF)), w_spec((NF, NF)),
        w_spec((NF, NF)), w_spec((1, NF)), w_spec((NF, NF)), w_spec((1, NF)),
        w_spec((NF, NF)), w_spec((1, NF)), w_spec((NF, NF)), w_spec((1, NF)),
    ]
    out_specs = [
        pl.BlockSpec((block, NF), lambda i: (i, 0)),
        pl.BlockSpec((block, NF), lambda i: (i, 0)),
        pl.BlockSpec((block, 3, NF), lambda i: (i, 0, 0)),
        pl.BlockSpec((block, 3, NF), lambda i: (i, 0, 0)),
    ]
    out_shape = [
        jax.ShapeDtypeStruct((a, NF), F32),
        jax.ShapeDtypeStruct((a, NF), F32),
        jax.ShapeDtypeStruct((a, 3, NF), F32),
        jax.ShapeDtypeStruct((a, 3, NF), F32),
    ]
    return pl.pallas_call(
        _main_body,
        grid=grid,
        in_specs=in_specs,
        out_specs=out_specs,
        out_shape=out_shape,
        interpret=interpret,
    )(inv, edge, dist, dv3, mask, dr_in, f_in, imn, gimn, gdr,
      w_ime, b_ime, wemc_row,
      w_emf1, b_emf1, w_emf2, b_emf2, w_eme1, w_eme2,
      w_esc1, b_esc1, w_esc2, b_esc2, w_isc1, b_isc1, w_isc2, b_isc2)


def kernel(invariant_node, invariant_edge, distances, distance_vector,
           neighbors, neighbor_mask, equivariant_node_F, equivariant_node_f,
           equivariant_node_dr,
           W_ime, b_ime, W_imn1, b_imn1, W_imn2, b_imn2, W_emc,
           W_emf1, b_emf1, W_emf2, b_emf2, W_esc1, b_esc1, W_esc2, b_esc2,
           W_eme1, W_eme2, W_isc1, b_isc1, W_isc2, b_isc2):
    B, A, Nn = neighbors.shape
    nb = invariant_edge.shape[-1]

    inv = invariant_node.reshape(A, NF)
    edge = invariant_edge.reshape(A * Nn, nb)
    dist = distances.reshape(A, Nn)
    dv3 = jnp.transpose(distance_vector.reshape(A, Nn, 3), (2, 0, 1))
    mask = neighbor_mask.reshape(A, Nn)
    dr_in = equivariant_node_dr.reshape(A, 3, NF)
    f_in = equivariant_node_f.reshape(A, 3, NF)

    def row(b):
        return b.reshape(1, NF)

    blk1 = 2000 if A % 2000 == 0 else A
    blk2 = 200 if A % 200 == 0 else A
    imn = _imn_call(inv, W_imn1, row(b_imn1), W_imn2, row(b_imn2), block=blk1)

    flat_nbr = neighbors.reshape(A * Nn)
    gimn = _gather_rows(imn, flat_nbr, NF, 128)
    gdr = _gather_rows(dr_in.reshape(A, 3 * NF), flat_nbr, 3 * NF, 128)
    gdr = gdr.reshape(A, Nn, 3 * NF)

    inv_out, F_out, f_out, dr_out = _main_call(
        inv, edge, dist, dv3, mask, dr_in, f_in, imn, gimn, gdr,
        W_ime, row(b_ime), W_emc.reshape(1, NF),
        W_emf1, row(b_emf1), W_emf2, row(b_emf2), W_eme1, W_eme2,
        W_esc1, row(b_esc1), W_esc2, row(b_esc2),
        W_isc1, row(b_isc1), W_isc2, row(b_isc2),
        block=blk2)

    F_final = equivariant_node_F.reshape(A, 3) + F_out[:, :3]
    return (inv_out.reshape(B, A, NF),
            F_final.reshape(B, A, 3),
            f_out.reshape(B, A, 3, NF),
            dr_out.reshape(B, A, 3, NF))
